# Initial kernel scaffold; baseline (speedup 1.0000x reference)
import jax
import jax.numpy as jnp
from jax.experimental import pallas as pl


def kernel(x, pos, batch, knn_idx, W0a, b0a, W0b, b0b, W1a, b1a, W1b, b1b, W2a, b2a, W2b, b2b, Wg, bg, Wh1, bh1, Wh2, bh2, Wh3, bh3, Wh4, bh4):
    raise NotImplementedError("write your pallas kernel here")


# speedup vs baseline: 1.1644x; 1.1644x over previous
import functools

import jax
import jax.numpy as jnp
from jax import lax
from jax.experimental import pallas as pl
from jax.experimental.pallas import tpu as pltpu
from jax.experimental.pallas import tpu_sc as plsc

_PREC = lax.Precision.HIGHEST

N = 10000
K = 20
NUM_GRAPHS = 4
BN = 1000
NB = N // BN
CHUNK = 128
NW = 32
E = K * N
NTASK = (E + CHUNK - 1) // CHUNK
TAIL = E - (NTASK - 1) * CHUNK
ITERS = (NTASK + NW - 1) // NW


def _sc_gather(table, idx2):
    C = table.shape[1]
    mesh = plsc.VectorSubcoreMesh(core_axis_name="c", subcore_axis_name="s")

    @functools.partial(
        pl.kernel,
        mesh=mesh,
        compiler_params=pltpu.CompilerParams(use_tc_tiling_on_sc=False),
        out_type=jax.ShapeDtypeStruct((E, C), jnp.float32),
        scratch_types=[
            pltpu.VMEM((CHUNK,), jnp.int32),
            pltpu.VMEM((CHUNK, C), jnp.float32),
            pltpu.SemaphoreType.DMA,
        ],
    )
    def gather_kernel(table_hbm, idx_hbm, out_hbm, idx_v, rows_v, sem):
        wid = lax.axis_index("s") * 2 + lax.axis_index("c")

        def body(j, carry):
            t = j * NW + wid

            @pl.when(t < NTASK)
            def _():
                pltpu.sync_copy(idx_hbm.at[t], idx_v)
                pltpu.async_copy(table_hbm.at[idx_v], rows_v, sem).wait()

                @pl.when(t < NTASK - 1)
                def _():
                    pltpu.sync_copy(rows_v, out_hbm.at[pl.ds(t * CHUNK, CHUNK)])

                @pl.when(t == NTASK - 1)
                def _():
                    pltpu.sync_copy(
                        rows_v.at[pl.ds(0, TAIL)],
                        out_hbm.at[pl.ds((NTASK - 1) * CHUNK, TAIL)],
                    )

            return carry

        lax.fori_loop(0, ITERS, body, 0)

    return gather_kernel(table, idx2)


def _conv_layer(f, g3, wd, wab, ba, wb, bb):
    C = f.shape[1]
    Co = wb.shape[0]

    def body(f_ref, g_ref, wd_ref, wab_ref, ba_ref, wb_ref, bb_ref, o_ref):
        fb = f_ref[...]
        base = jnp.dot(fb, wd_ref[...], preferred_element_type=jnp.float32, precision=_PREC) + ba_ref[...]
        wab_v = wab_ref[...]
        wb_v = wb_ref[...]
        bb_v = bb_ref[...]
        acc = jnp.zeros((BN, Co), jnp.float32)
        for k in range(K):
            h = jnp.maximum(
                jnp.dot(g_ref[k], wab_v, preferred_element_type=jnp.float32, precision=_PREC) + base, 0.0)
            h = jnp.maximum(
                jnp.dot(h, wb_v, preferred_element_type=jnp.float32, precision=_PREC) + bb_v, 0.0)
            acc = jnp.maximum(acc, h)
        o_ref[...] = acc

    return pl.pallas_call(
        body,
        grid=(NB,),
        in_specs=[
            pl.BlockSpec((BN, C), lambda i: (i, 0)),
            pl.BlockSpec((K, BN, C), lambda i: (0, i, 0)),
            pl.BlockSpec((C, Co), lambda i: (0, 0)),
            pl.BlockSpec((C, Co), lambda i: (0, 0)),
            pl.BlockSpec((1, Co), lambda i: (0, 0)),
            pl.BlockSpec((Co, Co), lambda i: (0, 0)),
            pl.BlockSpec((1, Co), lambda i: (0, 0)),
        ],
        out_specs=pl.BlockSpec((BN, Co), lambda i: (i, 0)),
        out_shape=jax.ShapeDtypeStruct((N, Co), jnp.float32),
    )(f, g3, wd, wab, ba, wb, bb)


def _head_pool(batch2, f1, f2, f3, wg1, wg2, wg3, bg):
    def body(b_ref, f1_ref, f2_ref, f3_ref, wg1_ref, wg2_ref, wg3_ref,
             bg_ref, sums_ref, cnt_ref):
        i = pl.program_id(0)

        @pl.when(i == 0)
        def _():
            sums_ref[...] = jnp.zeros_like(sums_ref)
            cnt_ref[...] = jnp.zeros_like(cnt_ref)

        b = b_ref[...]
        seg = lax.broadcasted_iota(jnp.int32, (1, 8), 1)
        oh = (b == seg).astype(jnp.float32)
        g = jnp.dot(f1_ref[...], wg1_ref[...], preferred_element_type=jnp.float32, precision=_PREC)
        g = g + jnp.dot(f2_ref[...], wg2_ref[...], preferred_element_type=jnp.float32, precision=_PREC)
        g = g + jnp.dot(f3_ref[...], wg3_ref[...], preferred_element_type=jnp.float32, precision=_PREC)
        g = jnp.maximum(g + bg_ref[...], 0.0)
        sums_ref[...] += lax.dot_general(
            oh, g, (((0,), (0,)), ((), ())), preferred_element_type=jnp.float32, precision=_PREC)
        cnt_ref[...] += lax.dot_general(
            oh, jnp.ones((BN, 128), jnp.float32), (((0,), (0,)), ((), ())),
            preferred_element_type=jnp.float32, precision=_PREC)

    return pl.pallas_call(
        body,
        grid=(NB,),
        in_specs=[
            pl.BlockSpec((BN, 1), lambda i: (i, 0)),
            pl.BlockSpec((BN, 64), lambda i: (i, 0)),
            pl.BlockSpec((BN, 128), lambda i: (i, 0)),
            pl.BlockSpec((BN, 256), lambda i: (i, 0)),
            pl.BlockSpec((64, 1024), lambda i: (0, 0)),
            pl.BlockSpec((128, 1024), lambda i: (0, 0)),
            pl.BlockSpec((256, 1024), lambda i: (0, 0)),
            pl.BlockSpec((1, 1024), lambda i: (0, 0)),
        ],
        out_specs=[
            pl.BlockSpec((8, 1024), lambda i: (0, 0)),
            pl.BlockSpec((8, 128), lambda i: (0, 0)),
        ],
        out_shape=[
            jax.ShapeDtypeStruct((8, 1024), jnp.float32),
            jax.ShapeDtypeStruct((8, 128), jnp.float32),
        ],
    )(batch2, f1, f2, f3, wg1, wg2, wg3, bg)


def _head_final(batch2, f1, f2, f3, sums, cnt,
                wh1g, wh11, wh12, wh13, bh1, wh2, bh2, wh3, bh3, wh4, bh4):
    def body(b_ref, f1_ref, f2_ref, f3_ref, sums_ref, cnt_ref,
             wh1g_ref, wh11_ref, wh12_ref, wh13_ref, bh1_ref,
             wh2_ref, bh2_ref, wh3_ref, bh3_ref, wh4_ref, bh4_ref, o_ref):
        cnt1 = cnt_ref[...][:, 0:1]
        mean = sums_ref[...] * (1.0 / jnp.maximum(cnt1, 1.0))
        m1 = jnp.dot(mean, wh1g_ref[...], preferred_element_type=jnp.float32, precision=_PREC)
        m1 = m1 + bh1_ref[...]
        b = b_ref[...]
        seg = lax.broadcasted_iota(jnp.int32, (1, 8), 1)
        oh = (b == seg).astype(jnp.float32)
        h = jnp.dot(oh, m1, preferred_element_type=jnp.float32, precision=_PREC)
        h = h + jnp.dot(f1_ref[...], wh11_ref[...], preferred_element_type=jnp.float32, precision=_PREC)
        h = h + jnp.dot(f2_ref[...], wh12_ref[...], preferred_element_type=jnp.float32, precision=_PREC)
        h = h + jnp.dot(f3_ref[...], wh13_ref[...], preferred_element_type=jnp.float32, precision=_PREC)
        h = jnp.maximum(h, 0.0)
        h = jnp.maximum(
            jnp.dot(h, wh2_ref[...], preferred_element_type=jnp.float32, precision=_PREC) + bh2_ref[...], 0.0)
        h = jnp.dot(h, wh3_ref[...], preferred_element_type=jnp.float32, precision=_PREC) + bh3_ref[...]
        h = jnp.where(h > 0, h, 0.2 * h)
        o_ref[...] = (
            jnp.dot(h, wh4_ref[...], preferred_element_type=jnp.float32, precision=_PREC) + bh4_ref[...])

    return pl.pallas_call(
        body,
        grid=(NB,),
        in_specs=[
            pl.BlockSpec((BN, 1), lambda i: (i, 0)),
            pl.BlockSpec((BN, 64), lambda i: (i, 0)),
            pl.BlockSpec((BN, 128), lambda i: (i, 0)),
            pl.BlockSpec((BN, 256), lambda i: (i, 0)),
            pl.BlockSpec((8, 1024), lambda i: (0, 0)),
            pl.BlockSpec((8, 128), lambda i: (0, 0)),
            pl.BlockSpec((1024, 256), lambda i: (0, 0)),
            pl.BlockSpec((64, 256), lambda i: (0, 0)),
            pl.BlockSpec((128, 256), lambda i: (0, 0)),
            pl.BlockSpec((256, 256), lambda i: (0, 0)),
            pl.BlockSpec((1, 256), lambda i: (0, 0)),
            pl.BlockSpec((256, 256), lambda i: (0, 0)),
            pl.BlockSpec((1, 256), lambda i: (0, 0)),
            pl.BlockSpec((256, 128), lambda i: (0, 0)),
            pl.BlockSpec((1, 128), lambda i: (0, 0)),
            pl.BlockSpec((128, 1), lambda i: (0, 0)),
            pl.BlockSpec((1, 1), lambda i: (0, 0)),
        ],
        out_specs=pl.BlockSpec((BN, 1), lambda i: (i, 0)),
        out_shape=jax.ShapeDtypeStruct((N, 1), jnp.float32),
    )(batch2, f1, f2, f3, sums, cnt,
      wh1g, wh11, wh12, wh13, bh1, wh2, bh2, wh3, bh3, wh4, bh4)


def kernel(x, pos, batch, knn_idx,
           W0a, b0a, W0b, b0b,
           W1a, b1a, W1b, b1b,
           W2a, b2a, W2b, b2b,
           Wg, bg, Wh1, bh1, Wh2, bh2, Wh3, bh3, Wh4, bh4):
    knn_t = jnp.transpose(knn_idx.astype(jnp.int32)).reshape(-1)
    idx2 = jnp.concatenate(
        [knn_t, jnp.zeros((NTASK * CHUNK - E,), jnp.int32)]).reshape(NTASK, CHUNK)
    batch2 = batch.astype(jnp.int32).reshape(N, 1)

    f0 = jnp.concatenate([pos, x, jnp.zeros((N, 2), jnp.float32)], axis=1)
    zpad = jnp.zeros((2, W0a.shape[1]), jnp.float32)
    wd0 = jnp.concatenate([W0a[:6] - W0a[6:], zpad], axis=0)
    wab0 = jnp.concatenate([W0a[6:], zpad], axis=0)
    wd1, wab1 = W1a[:64] - W1a[64:], W1a[64:]
    wd2, wab2 = W2a[:128] - W2a[128:], W2a[128:]
    r1 = lambda v: v.reshape(1, -1)

    g0 = _sc_gather(f0, idx2).reshape(K, N, 8)
    f1 = _conv_layer(f0, g0, wd0, wab0, r1(b0a), W0b, r1(b0b))
    g1 = _sc_gather(f1, idx2).reshape(K, N, 64)
    f2 = _conv_layer(f1, g1, wd1, wab1, r1(b1a), W1b, r1(b1b))
    g2 = _sc_gather(f2, idx2).reshape(K, N, 128)
    f3 = _conv_layer(f2, g2, wd2, wab2, r1(b2a), W2b, r1(b2b))

    sums, cnt = _head_pool(batch2, f1, f2, f3,
                           Wg[:64], Wg[64:192], Wg[192:], r1(bg))
    out = _head_final(batch2, f1, f2, f3, sums, cnt,
                      Wh1[:1024], Wh1[1024:1088], Wh1[1088:1216], Wh1[1216:],
                      r1(bh1), Wh2, r1(bh2), Wh3, r1(bh3), Wh4, r1(bh4))
    return out


# speedup vs baseline: 1.8559x; 1.5938x over previous
import functools

import jax
import jax.numpy as jnp
from jax import lax
from jax.experimental import pallas as pl
from jax.experimental.pallas import tpu as pltpu
from jax.experimental.pallas import tpu_sc as plsc

N = 10000
K = 20
NUM_GRAPHS = 4
BN = 1000
NB = N // BN
CHUNK = 128
NW = 32
E = K * N
NTASK = (E + CHUNK - 1) // CHUNK
TAIL = E - (NTASK - 1) * CHUNK
ITERS = (NTASK + NW - 1) // NW

_F32 = jnp.float32
_BF16 = jnp.bfloat16


def _split(x):
    xh = x.astype(_BF16)
    xl = (x - xh.astype(_F32)).astype(_BF16)
    return xh, xl


def _dot1(a, b):
    return jnp.dot(a, b, preferred_element_type=_F32)


def _dot3(a2, b2):
    ah, al = a2
    bh, bl = b2
    return _dot1(ah, bh) + _dot1(ah, bl) + _dot1(al, bh)


def _sc_gather(table, idx2):
    C = table.shape[1]
    mesh = plsc.VectorSubcoreMesh(core_axis_name="c", subcore_axis_name="s")

    @functools.partial(
        pl.kernel,
        mesh=mesh,
        compiler_params=pltpu.CompilerParams(use_tc_tiling_on_sc=False),
        out_type=jax.ShapeDtypeStruct((E, C), jnp.float32),
        scratch_types=[
            pltpu.VMEM((CHUNK,), jnp.int32),
            pltpu.VMEM((CHUNK, C), jnp.float32),
            pltpu.SemaphoreType.DMA,
        ],
    )
    def gather_kernel(table_hbm, idx_hbm, out_hbm, idx_v, rows_v, sem):
        wid = lax.axis_index("s") * 2 + lax.axis_index("c")

        def body(j, carry):
            t = j * NW + wid

            @pl.when(t < NTASK)
            def _():
                pltpu.sync_copy(idx_hbm.at[t], idx_v)
                pltpu.async_copy(table_hbm.at[idx_v], rows_v, sem).wait()

                @pl.when(t < NTASK - 1)
                def _():
                    pltpu.sync_copy(rows_v, out_hbm.at[pl.ds(t * CHUNK, CHUNK)])

                @pl.when(t == NTASK - 1)
                def _():
                    pltpu.sync_copy(
                        rows_v.at[pl.ds(0, TAIL)],
                        out_hbm.at[pl.ds((NTASK - 1) * CHUNK, TAIL)],
                    )

            return carry

        lax.fori_loop(0, ITERS, body, 0)

    return gather_kernel(table, idx2)


def _conv_layer(f, g3, wd, wab, ba, wb, bb):
    C = f.shape[1]
    Co = wb.shape[0]

    def body(f_ref, g_ref, wd_ref, wab_ref, ba_ref, wb_ref, bb_ref, o_ref):
        base = _dot3(_split(f_ref[...]), _split(wd_ref[...])) + ba_ref[...]
        wab2 = _split(wab_ref[...])
        wb2 = _split(wb_ref[...])
        bb_v = bb_ref[...]
        acc = jnp.zeros((BN, Co), jnp.float32)
        for k in range(K):
            h = jnp.maximum(_dot3(_split(g_ref[k]), wab2) + base, 0.0)
            h = jnp.maximum(_dot3(_split(h), wb2) + bb_v, 0.0)
            acc = jnp.maximum(acc, h)
        o_ref[...] = acc

    return pl.pallas_call(
        body,
        grid=(NB,),
        in_specs=[
            pl.BlockSpec((BN, C), lambda i: (i, 0)),
            pl.BlockSpec((K, BN, C), lambda i: (0, i, 0)),
            pl.BlockSpec((C, Co), lambda i: (0, 0)),
            pl.BlockSpec((C, Co), lambda i: (0, 0)),
            pl.BlockSpec((1, Co), lambda i: (0, 0)),
            pl.BlockSpec((Co, Co), lambda i: (0, 0)),
            pl.BlockSpec((1, Co), lambda i: (0, 0)),
        ],
        out_specs=pl.BlockSpec((BN, Co), lambda i: (i, 0)),
        out_shape=jax.ShapeDtypeStruct((N, Co), jnp.float32),
    )(f, g3, wd, wab, ba, wb, bb)


def _head_pool(batch2, f1, f2, f3, wg1, wg2, wg3, bg):
    def body(b_ref, f1_ref, f2_ref, f3_ref, wg1_ref, wg2_ref, wg3_ref,
             bg_ref, sums_ref, cnt_ref):
        i = pl.program_id(0)

        @pl.when(i == 0)
        def _():
            sums_ref[...] = jnp.zeros_like(sums_ref)
            cnt_ref[...] = jnp.zeros_like(cnt_ref)

        b = b_ref[...]
        seg = lax.broadcasted_iota(jnp.int32, (1, 8), 1)
        oh = (b == seg).astype(_BF16)
        g = _dot3(_split(f1_ref[...]), _split(wg1_ref[...]))
        g = g + _dot3(_split(f2_ref[...]), _split(wg2_ref[...]))
        g = g + _dot3(_split(f3_ref[...]), _split(wg3_ref[...]))
        g = jnp.maximum(g + bg_ref[...], 0.0)
        gh, gl = _split(g)
        dn = (((0,), (0,)), ((), ()))
        sums_ref[...] += (
            lax.dot_general(oh, gh, dn, preferred_element_type=_F32)
            + lax.dot_general(oh, gl, dn, preferred_element_type=_F32))
        cnt_ref[...] += lax.dot_general(
            oh, jnp.ones((BN, 128), _BF16), dn, preferred_element_type=_F32)

    return pl.pallas_call(
        body,
        grid=(NB,),
        in_specs=[
            pl.BlockSpec((BN, 1), lambda i: (i, 0)),
            pl.BlockSpec((BN, 64), lambda i: (i, 0)),
            pl.BlockSpec((BN, 128), lambda i: (i, 0)),
            pl.BlockSpec((BN, 256), lambda i: (i, 0)),
            pl.BlockSpec((64, 1024), lambda i: (0, 0)),
            pl.BlockSpec((128, 1024), lambda i: (0, 0)),
            pl.BlockSpec((256, 1024), lambda i: (0, 0)),
            pl.BlockSpec((1, 1024), lambda i: (0, 0)),
        ],
        out_specs=[
            pl.BlockSpec((8, 1024), lambda i: (0, 0)),
            pl.BlockSpec((8, 128), lambda i: (0, 0)),
        ],
        out_shape=[
            jax.ShapeDtypeStruct((8, 1024), jnp.float32),
            jax.ShapeDtypeStruct((8, 128), jnp.float32),
        ],
    )(batch2, f1, f2, f3, wg1, wg2, wg3, bg)


def _head_final(batch2, f1, f2, f3, sums, cnt,
                wh1g, wh11, wh12, wh13, bh1, wh2, bh2, wh3, bh3, wh4, bh4):
    def body(b_ref, f1_ref, f2_ref, f3_ref, sums_ref, cnt_ref,
             wh1g_ref, wh11_ref, wh12_ref, wh13_ref, bh1_ref,
             wh2_ref, bh2_ref, wh3_ref, bh3_ref, wh4_ref, bh4_ref, o_ref):
        cnt1 = cnt_ref[...][:, 0:1]
        mean = sums_ref[...] * (1.0 / jnp.maximum(cnt1, 1.0))
        m1 = _dot3(_split(mean), _split(wh1g_ref[...])) + bh1_ref[...]
        b = b_ref[...]
        seg = lax.broadcasted_iota(jnp.int32, (1, 8), 1)
        oh = (b == seg).astype(_BF16)
        m1h, m1l = _split(m1)
        h = _dot1(oh, m1h) + _dot1(oh, m1l)
        h = h + _dot3(_split(f1_ref[...]), _split(wh11_ref[...]))
        h = h + _dot3(_split(f2_ref[...]), _split(wh12_ref[...]))
        h = h + _dot3(_split(f3_ref[...]), _split(wh13_ref[...]))
        h = jnp.maximum(h, 0.0)
        h = jnp.maximum(
            _dot3(_split(h), _split(wh2_ref[...])) + bh2_ref[...], 0.0)
        h = _dot3(_split(h), _split(wh3_ref[...])) + bh3_ref[...]
        h = jnp.where(h > 0, h, 0.2 * h)
        o_ref[...] = _dot3(_split(h), _split(wh4_ref[...])) + bh4_ref[...]

    return pl.pallas_call(
        body,
        grid=(NB,),
        in_specs=[
            pl.BlockSpec((BN, 1), lambda i: (i, 0)),
            pl.BlockSpec((BN, 64), lambda i: (i, 0)),
            pl.BlockSpec((BN, 128), lambda i: (i, 0)),
            pl.BlockSpec((BN, 256), lambda i: (i, 0)),
            pl.BlockSpec((8, 1024), lambda i: (0, 0)),
            pl.BlockSpec((8, 128), lambda i: (0, 0)),
            pl.BlockSpec((1024, 256), lambda i: (0, 0)),
            pl.BlockSpec((64, 256), lambda i: (0, 0)),
            pl.BlockSpec((128, 256), lambda i: (0, 0)),
            pl.BlockSpec((256, 256), lambda i: (0, 0)),
            pl.BlockSpec((1, 256), lambda i: (0, 0)),
            pl.BlockSpec((256, 256), lambda i: (0, 0)),
            pl.BlockSpec((1, 256), lambda i: (0, 0)),
            pl.BlockSpec((256, 128), lambda i: (0, 0)),
            pl.BlockSpec((1, 128), lambda i: (0, 0)),
            pl.BlockSpec((128, 1), lambda i: (0, 0)),
            pl.BlockSpec((1, 1), lambda i: (0, 0)),
        ],
        out_specs=pl.BlockSpec((BN, 1), lambda i: (i, 0)),
        out_shape=jax.ShapeDtypeStruct((N, 1), jnp.float32),
    )(batch2, f1, f2, f3, sums, cnt,
      wh1g, wh11, wh12, wh13, bh1, wh2, bh2, wh3, bh3, wh4, bh4)


def kernel(x, pos, batch, knn_idx,
           W0a, b0a, W0b, b0b,
           W1a, b1a, W1b, b1b,
           W2a, b2a, W2b, b2b,
           Wg, bg, Wh1, bh1, Wh2, bh2, Wh3, bh3, Wh4, bh4):
    knn_t = jnp.transpose(knn_idx.astype(jnp.int32)).reshape(-1)
    idx2 = jnp.concatenate(
        [knn_t, jnp.zeros((NTASK * CHUNK - E,), jnp.int32)]).reshape(NTASK, CHUNK)
    batch2 = batch.astype(jnp.int32).reshape(N, 1)

    f0 = jnp.concatenate([pos, x, jnp.zeros((N, 2), jnp.float32)], axis=1)
    zpad = jnp.zeros((2, W0a.shape[1]), jnp.float32)
    wd0 = jnp.concatenate([W0a[:6] - W0a[6:], zpad], axis=0)
    wab0 = jnp.concatenate([W0a[6:], zpad], axis=0)
    wd1, wab1 = W1a[:64] - W1a[64:], W1a[64:]
    wd2, wab2 = W2a[:128] - W2a[128:], W2a[128:]
    r1 = lambda v: v.reshape(1, -1)

    g0 = _sc_gather(f0, idx2).reshape(K, N, 8)
    f1 = _conv_layer(f0, g0, wd0, wab0, r1(b0a), W0b, r1(b0b))
    g1 = _sc_gather(f1, idx2).reshape(K, N, 64)
    f2 = _conv_layer(f1, g1, wd1, wab1, r1(b1a), W1b, r1(b1b))
    g2 = _sc_gather(f2, idx2).reshape(K, N, 128)
    f3 = _conv_layer(f2, g2, wd2, wab2, r1(b2a), W2b, r1(b2b))

    sums, cnt = _head_pool(batch2, f1, f2, f3,
                           Wg[:64], Wg[64:192], Wg[192:], r1(bg))
    out = _head_final(batch2, f1, f2, f3, sums, cnt,
                      Wh1[:1024], Wh1[1024:1088], Wh1[1088:1216], Wh1[1216:],
                      r1(bh1), Wh2, r1(bh2), Wh3, r1(bh3), Wh4, r1(bh4))
    return out
